# Initial kernel scaffold; baseline (speedup 1.0000x reference)
import jax
import jax.numpy as jnp
from jax.experimental import pallas as pl


def kernel(x, table):
    raise NotImplementedError("write your pallas kernel here")


# speedup vs baseline: 2.6047x; 2.6047x over previous
import functools
import jax
import jax.numpy as jnp
from jax import lax
from jax.experimental import pallas as pl
from jax.experimental.pallas import tpu as pltpu
from jax.experimental.pallas import tpu_sc as plsc

PAD = 4
L = 16


@functools.lru_cache(maxsize=None)
def _make_lookup(B, V, D, n_workers):
    assert B % n_workers == 0
    b_per_w = B // n_workers
    assert b_per_w % L == 0

    chunk = b_per_w
    while chunk * D * 4 > 220 * 1024:
        chunk //= 2
    assert b_per_w % chunk == 0 and chunk % 8 == 0
    n_chunks = b_per_w // chunk

    mesh = plsc.VectorSubcoreMesh(core_axis_name="c", subcore_axis_name="s")

    @functools.partial(
        pl.kernel,
        out_type=jax.ShapeDtypeStruct((B, D), jnp.float32),
        mesh=mesh,
        compiler_params=pltpu.CompilerParams(use_tc_tiling_on_sc=False),
        scratch_types=[
            pltpu.VMEM((b_per_w,), jnp.int32),
            pltpu.VMEM((chunk, D), jnp.float32),
            pltpu.VMEM((chunk, D), jnp.float32),
            pltpu.SemaphoreType.DMA,
            pltpu.SemaphoreType.DMA,
            pltpu.SemaphoreType.DMA,
            pltpu.SemaphoreType.DMA,
        ],
    )
    def lookup(table_hbm, idx_hbm, out_hbm, idx_v, rows0, rows1, g0, g1, s0, s1):
        wid = lax.axis_index("s") * 2 + lax.axis_index("c")
        base = wid * b_per_w

        pltpu.sync_copy(idx_hbm.at[pl.ds(base, b_per_w)], idx_v)

        pad_v = jnp.full((L,), PAD, jnp.int32)
        zrow_v = jnp.full((L,), V - 1, jnp.int32)

        def remap(i):
            v = idx_v[pl.ds(i * L, L)]
            idx_v[pl.ds(i * L, L)] = jnp.where(v == pad_v, zrow_v, v)

        pl.loop(0, b_per_w // L)(remap)

        bufs = ((rows0, g0, s0), (rows1, g1, s1))
        gathers = [None, None]
        writes = [None, None]

        def gather(c):
            rows, g, _ = bufs[c % 2]
            if writes[c % 2] is not None:
                writes[c % 2].wait()
                writes[c % 2] = None
            gathers[c % 2] = pltpu.async_copy(
                table_hbm.at[idx_v.at[pl.ds(c * chunk, chunk)]], rows, g
            )

        def writeout(c):
            rows, _, s = bufs[c % 2]
            gathers[c % 2].wait()
            writes[c % 2] = pltpu.async_copy(
                rows, out_hbm.at[pl.ds(base + c * chunk, chunk)], s
            )

        gather(0)
        for c in range(1, n_chunks):
            gather(c)
            writeout(c - 1)
        writeout(n_chunks - 1)
        for w in writes:
            if w is not None:
                w.wait()

    return lookup


def kernel(x, table):
    B_, S = x.shape
    V, D = table.shape
    table_ext = jnp.concatenate([table, jnp.zeros((1, D), table.dtype)], axis=0)
    idx = x.reshape(-1).astype(jnp.int32)
    out = _make_lookup(idx.shape[0], V + 1, D, 32)(table_ext, idx)
    return out.reshape(B_, S, D)


# speedup vs baseline: 10.5150x; 4.0370x over previous
import functools
import jax
import jax.numpy as jnp
from jax import lax
from jax.experimental import pallas as pl
from jax.experimental.pallas import tpu as pltpu
from jax.experimental.pallas import tpu_sc as plsc

PAD = 4
L = 16


@functools.lru_cache(maxsize=None)
def _make_lookup(V, D, S, Bb):
    CH = 4096
    NB = Bb // CH
    steps = S * NB
    assert Bb % CH == 0 and CH % L == 0 and steps % 2 == 0

    mesh = plsc.VectorSubcoreMesh(core_axis_name="c", subcore_axis_name="s")

    @functools.partial(
        pl.kernel,
        out_type=jax.ShapeDtypeStruct((D, S, Bb), jnp.float32),
        mesh=mesh,
        compiler_params=pltpu.CompilerParams(
            use_tc_tiling_on_sc=True, needs_layout_passes=False
        ),
        scratch_types=[
            pltpu.VMEM((V,), jnp.float32),
            pltpu.VMEM((CH,), jnp.int32),
            pltpu.VMEM((CH,), jnp.int32),
            pltpu.VMEM((CH,), jnp.float32),
            pltpu.VMEM((CH,), jnp.float32),
            pltpu.SemaphoreType.DMA,
            pltpu.SemaphoreType.DMA,
            pltpu.SemaphoreType.DMA,
            pltpu.SemaphoreType.DMA,
        ],
    )
    def lookup(tT_hbm, xT_hbm, out_hbm, plane_v, i0, i1, o0, o1,
               si0, si1, so0, so1):
        wid = lax.axis_index("s") * 2 + lax.axis_index("c")

        @pl.when(wid < D)
        def _():
            d = wid
            pltpu.sync_copy(tT_hbm.at[d], plane_v)
            lane = lax.iota(jnp.int32, L)
            plane_v[pl.ds(0, L)] = jnp.where(
                lane == PAD, 0.0, plane_v[pl.ds(0, L)]
            )

            ibufs, obufs = (i0, i1), (o0, o1)
            isems, osems = (si0, si1), (so0, so1)

            def start_idx(j, b):
                srow = j // NB
                bcol = (j % NB) * CH
                pltpu.make_async_copy(
                    xT_hbm.at[srow, pl.ds(bcol, CH)], ibufs[b], isems[b]
                ).start()

            def wait_idx(b):
                pltpu.make_async_copy(
                    xT_hbm.at[0, pl.ds(0, CH)], ibufs[b], isems[b]
                ).wait()

            def start_out(j, b):
                srow = j // NB
                bcol = (j % NB) * CH
                pltpu.make_async_copy(
                    obufs[b], out_hbm.at[d, srow, pl.ds(bcol, CH)], osems[b]
                ).start()

            def wait_out(b):
                pltpu.make_async_copy(
                    obufs[b], out_hbm.at[d, 0, pl.ds(0, CH)], osems[b]
                ).wait()

            start_idx(0, 0)
            start_idx(1, 1)

            def body(j):
                for b in range(2):
                    jj = j + b
                    wait_idx(b)

                    @pl.when(jj >= 2)
                    def _():
                        wait_out(b)

                    def grp(g):
                        iv = ibufs[b][pl.ds(g * L, L)]
                        obufs[b][pl.ds(g * L, L)] = plsc.load_gather(
                            plane_v, [iv]
                        )

                    pl.loop(0, CH // L)(grp)
                    start_out(jj, b)

                    @pl.when(jj + 2 < steps)
                    def _():
                        start_idx(jj + 2, b)

            pl.loop(0, steps, step=2)(body)
            wait_out(0)
            wait_out(1)

    return lookup


def kernel(x, table):
    B_, S = x.shape
    V, D = table.shape
    out3 = _make_lookup(V, D, S, B_)(table.T, x.T)
    return jnp.transpose(out3, (2, 1, 0))


# speedup vs baseline: 14.1647x; 1.3471x over previous
import functools
import jax
import jax.numpy as jnp
from jax import lax
from jax.experimental import pallas as pl
from jax.experimental.pallas import tpu as pltpu
from jax.experimental.pallas import tpu_sc as plsc

PAD = 4
L = 16


@functools.lru_cache(maxsize=None)
def _make_lookup(V, D, S, Bb):
    CH = 4096
    NB = Bb // CH
    steps = S * NB
    assert Bb % CH == 0 and CH % L == 0 and steps % 2 == 0

    mesh = plsc.VectorSubcoreMesh(core_axis_name="c", subcore_axis_name="s")

    @functools.partial(
        pl.kernel,
        out_type=jax.ShapeDtypeStruct((D, S, Bb), jnp.float32),
        mesh=mesh,
        compiler_params=pltpu.CompilerParams(
            use_tc_tiling_on_sc=True, needs_layout_passes=False
        ),
        scratch_types=[
            pltpu.VMEM((V,), jnp.float32),
            pltpu.VMEM((CH,), jnp.int32),
            pltpu.VMEM((CH,), jnp.int32),
            pltpu.VMEM((CH,), jnp.float32),
            pltpu.VMEM((CH,), jnp.float32),
            pltpu.SemaphoreType.DMA,
            pltpu.SemaphoreType.DMA,
            pltpu.SemaphoreType.DMA,
            pltpu.SemaphoreType.DMA,
        ],
    )
    def lookup(tT_hbm, xT_hbm, out_hbm, plane_v, i0, i1, o0, o1,
               si0, si1, so0, so1):
        wid = lax.axis_index("s") * 2 + lax.axis_index("c")

        @pl.when(wid < D)
        def _():
            d = wid
            pltpu.sync_copy(tT_hbm.at[d], plane_v)
            lane = lax.iota(jnp.int32, L)
            plane_v[pl.ds(0, L)] = jnp.where(
                lane == PAD, 0.0, plane_v[pl.ds(0, L)]
            )

            ibufs, obufs = (i0, i1), (o0, o1)
            isems, osems = (si0, si1), (so0, so1)

            def start_idx(j, b):
                srow = j // NB
                bcol = (j % NB) * CH
                pltpu.make_async_copy(
                    xT_hbm.at[srow, pl.ds(bcol, CH)], ibufs[b], isems[b]
                ).start()

            def wait_idx(b):
                pltpu.make_async_copy(
                    xT_hbm.at[0, pl.ds(0, CH)], ibufs[b], isems[b]
                ).wait()

            def start_out(j, b):
                srow = j // NB
                bcol = (j % NB) * CH
                pltpu.make_async_copy(
                    obufs[b], out_hbm.at[d, srow, pl.ds(bcol, CH)], osems[b]
                ).start()

            def wait_out(b):
                pltpu.make_async_copy(
                    obufs[b], out_hbm.at[d, 0, pl.ds(0, CH)], osems[b]
                ).wait()

            start_idx(0, 0)
            start_idx(1, 1)

            def body(j):
                for b in range(2):
                    jj = j + b
                    wait_idx(b)

                    @pl.when(jj >= 2)
                    def _():
                        wait_out(b)

                    def grp(g):
                        for u in range(8):
                            off = (g + u) * L
                            iv = ibufs[b][pl.ds(off, L)]
                            obufs[b][pl.ds(off, L)] = plsc.load_gather(
                                plane_v, [iv]
                            )

                    pl.loop(0, CH // L, step=8)(grp)
                    start_out(jj, b)

                    @pl.when(jj + 2 < steps)
                    def _():
                        start_idx(jj + 2, b)

            pl.loop(0, steps, step=2)(body)
            wait_out(0)
            wait_out(1)

    return lookup


def kernel(x, table):
    B_, S = x.shape
    V, D = table.shape
    out3 = _make_lookup(V, D, S, B_)(table.T, x.T)
    return jnp.transpose(out3, (2, 1, 0))


# speedup vs baseline: 14.2903x; 1.0089x over previous
import functools
import jax
import jax.numpy as jnp
from jax import lax
from jax.experimental import pallas as pl
from jax.experimental.pallas import tpu as pltpu
from jax.experimental.pallas import tpu_sc as plsc

PAD = 4
L = 16


@functools.lru_cache(maxsize=None)
def _make_lookup(V, D, S, Bb):
    CH = 4096
    NB = Bb // CH
    steps = S * NB
    assert Bb % CH == 0 and CH % L == 0 and steps % 2 == 0

    mesh = plsc.VectorSubcoreMesh(core_axis_name="c", subcore_axis_name="s")

    @functools.partial(
        pl.kernel,
        out_type=jax.ShapeDtypeStruct((D, S, Bb), jnp.float32),
        mesh=mesh,
        compiler_params=pltpu.CompilerParams(
            use_tc_tiling_on_sc=True, needs_layout_passes=False
        ),
        scratch_types=[
            pltpu.VMEM((V,), jnp.float32),
            pltpu.VMEM((CH,), jnp.int32),
            pltpu.VMEM((CH,), jnp.int32),
            pltpu.VMEM((CH,), jnp.float32),
            pltpu.VMEM((CH,), jnp.float32),
            pltpu.SemaphoreType.DMA,
            pltpu.SemaphoreType.DMA,
            pltpu.SemaphoreType.DMA,
            pltpu.SemaphoreType.DMA,
        ],
    )
    def lookup(tT_hbm, xT_hbm, out_hbm, plane_v, i0, i1, o0, o1,
               si0, si1, so0, so1):
        wid = lax.axis_index("s") * 2 + lax.axis_index("c")

        @pl.when(wid < D)
        def _():
            d = wid
            pltpu.sync_copy(tT_hbm.at[d], plane_v)
            lane = lax.iota(jnp.int32, L)
            plane_v[pl.ds(0, L)] = jnp.where(
                lane == PAD, 0.0, plane_v[pl.ds(0, L)]
            )

            ibufs, obufs = (i0, i1), (o0, o1)
            isems, osems = (si0, si1), (so0, so1)

            def start_idx(j, b):
                srow = j // NB
                bcol = (j % NB) * CH
                pltpu.make_async_copy(
                    xT_hbm.at[srow, pl.ds(bcol, CH)], ibufs[b], isems[b]
                ).start()

            def wait_idx(b):
                pltpu.make_async_copy(
                    xT_hbm.at[0, pl.ds(0, CH)], ibufs[b], isems[b]
                ).wait()

            def start_out(j, b):
                srow = j // NB
                bcol = (j % NB) * CH
                pltpu.make_async_copy(
                    obufs[b], out_hbm.at[d, srow, pl.ds(bcol, CH)], osems[b]
                ).start()

            def wait_out(b):
                pltpu.make_async_copy(
                    obufs[b], out_hbm.at[d, 0, pl.ds(0, CH)], osems[b]
                ).wait()

            start_idx(0, 0)
            start_idx(1, 1)

            def body(j):
                for b in range(2):
                    jj = j + b
                    wait_idx(b)

                    @pl.when(jj >= 2)
                    def _():
                        wait_out(b)

                    def grp(g):
                        for u in range(16):
                            off = (g + u) * L
                            iv = ibufs[b][pl.ds(off, L)]
                            obufs[b][pl.ds(off, L)] = plsc.load_gather(
                                plane_v, [iv]
                            )

                    pl.loop(0, CH // L, step=16)(grp)
                    start_out(jj, b)

                    @pl.when(jj + 2 < steps)
                    def _():
                        start_idx(jj + 2, b)

            pl.loop(0, steps, step=2)(body)
            wait_out(0)
            wait_out(1)

    return lookup


def kernel(x, table):
    B_, S = x.shape
    V, D = table.shape
    out3 = _make_lookup(V, D, S, B_)(table.T, x.T)
    return jnp.transpose(out3, (2, 1, 0))


# speedup vs baseline: 17.4594x; 1.2218x over previous
import functools
import jax
import jax.numpy as jnp
from jax import lax
from jax.experimental import pallas as pl
from jax.experimental.pallas import tpu as pltpu
from jax.experimental.pallas import tpu_sc as plsc

PAD = 4
L = 16
W = 32


@functools.lru_cache(maxsize=None)
def _make_lookup(V, D, S, Bb):
    CH = 2048
    NB = Bb // CH
    CPP = S * NB
    PW = (D * CPP) // W
    assert Bb % CH == 0 and CH % L == 0 and D * CPP == W * PW
    for w in range(W):
        r0 = (w * PW) % CPP
        l0 = min(CPP - r0, PW)
        assert l0 % 2 == 0 and (PW - l0) % 2 == 0

    mesh = plsc.VectorSubcoreMesh(core_axis_name="c", subcore_axis_name="s")

    @functools.partial(
        pl.kernel,
        out_type=jax.ShapeDtypeStruct((D, S, Bb), jnp.float32),
        mesh=mesh,
        compiler_params=pltpu.CompilerParams(
            use_tc_tiling_on_sc=True, needs_layout_passes=False
        ),
        scratch_types=[
            pltpu.VMEM((V,), jnp.float32),
            pltpu.VMEM((CH,), jnp.int32),
            pltpu.VMEM((CH,), jnp.int32),
            pltpu.VMEM((CH,), jnp.float32),
            pltpu.VMEM((CH,), jnp.float32),
            pltpu.SemaphoreType.DMA,
            pltpu.SemaphoreType.DMA,
            pltpu.SemaphoreType.DMA,
            pltpu.SemaphoreType.DMA,
        ],
    )
    def lookup(tT_hbm, xT_hbm, out_hbm, plane_v, i0, i1, o0, o1,
               si0, si1, so0, so1):
        wid = lax.axis_index("s") * 2 + lax.axis_index("c")
        start = wid * PW
        d0 = start // CPP
        r0 = start % CPP
        len0 = jnp.minimum(CPP - r0, PW)
        len1 = PW - len0

        ibufs, obufs = (i0, i1), (o0, o1)
        isems, osems = (si0, si1), (so0, so1)

        def run_segment(d, c0, steps):
            pltpu.sync_copy(tT_hbm.at[d], plane_v)
            lane = lax.iota(jnp.int32, L)
            plane_v[pl.ds(0, L)] = jnp.where(
                lane == PAD, 0.0, plane_v[pl.ds(0, L)]
            )

            def start_idx(j, b):
                c = c0 + j
                srow = c // NB
                bcol = (c % NB) * CH
                pltpu.make_async_copy(
                    xT_hbm.at[srow, pl.ds(bcol, CH)], ibufs[b], isems[b]
                ).start()

            def wait_idx(b):
                pltpu.make_async_copy(
                    xT_hbm.at[0, pl.ds(0, CH)], ibufs[b], isems[b]
                ).wait()

            def start_out(j, b):
                c = c0 + j
                srow = c // NB
                bcol = (c % NB) * CH
                pltpu.make_async_copy(
                    obufs[b], out_hbm.at[d, srow, pl.ds(bcol, CH)], osems[b]
                ).start()

            def wait_out(b):
                pltpu.make_async_copy(
                    obufs[b], out_hbm.at[d, 0, pl.ds(0, CH)], osems[b]
                ).wait()

            start_idx(0, 0)
            start_idx(1, 1)

            def body(j):
                for b in range(2):
                    jj = j + b
                    wait_idx(b)

                    @pl.when(jj >= 2)
                    def _():
                        wait_out(b)

                    def grp(g):
                        for u in range(16):
                            off = (g + u) * L
                            iv = ibufs[b][pl.ds(off, L)]
                            obufs[b][pl.ds(off, L)] = plsc.load_gather(
                                plane_v, [iv]
                            )

                    pl.loop(0, CH // L, step=16)(grp)
                    start_out(jj, b)

                    @pl.when(jj + 2 < steps)
                    def _():
                        start_idx(jj + 2, b)

            pl.loop(0, steps, step=2)(body)
            wait_out(0)
            wait_out(1)

        run_segment(d0, r0, len0)

        @pl.when(len1 > 0)
        def _():
            run_segment(d0 + 1, jnp.int32(0), len1)

    return lookup


def kernel(x, table):
    B_, S = x.shape
    V, D = table.shape
    out3 = _make_lookup(V, D, S, B_)(table.T, x.T)
    return jnp.transpose(out3, (2, 1, 0))


# speedup vs baseline: 19.6146x; 1.1234x over previous
import functools
import jax
import jax.numpy as jnp
from jax import lax
from jax.experimental import pallas as pl
from jax.experimental.pallas import tpu as pltpu
from jax.experimental.pallas import tpu_sc as plsc

PAD = 4
L = 16
W = 32
DEPTH = 4


@functools.lru_cache(maxsize=None)
def _make_lookup(V, D, S, Bb):
    CH = 2048
    NB = Bb // CH
    CPP = S * NB
    PW = (D * CPP) // W
    assert Bb % CH == 0 and CH % L == 0 and D * CPP == W * PW
    for w in range(W):
        r0 = (w * PW) % CPP
        l0 = min(CPP - r0, PW)
        assert l0 >= DEPTH and (PW - l0 == 0 or PW - l0 >= DEPTH)

    mesh = plsc.VectorSubcoreMesh(core_axis_name="c", subcore_axis_name="s")

    @functools.partial(
        pl.kernel,
        out_type=jax.ShapeDtypeStruct((D, S, Bb), jnp.float32),
        mesh=mesh,
        compiler_params=pltpu.CompilerParams(
            use_tc_tiling_on_sc=True, needs_layout_passes=False
        ),
        scratch_types=(
            [pltpu.VMEM((V,), jnp.float32)]
            + [pltpu.VMEM((CH,), jnp.int32)] * DEPTH
            + [pltpu.VMEM((CH,), jnp.float32)] * DEPTH
            + [pltpu.SemaphoreType.DMA] * (2 * DEPTH)
        ),
    )
    def lookup(tT_hbm, xT_hbm, out_hbm, plane_v, *bufs):
        ibufs = bufs[0:DEPTH]
        obufs = bufs[DEPTH:2 * DEPTH]
        isems = bufs[2 * DEPTH:3 * DEPTH]
        osems = bufs[3 * DEPTH:4 * DEPTH]

        wid = lax.axis_index("s") * 2 + lax.axis_index("c")
        start = wid * PW
        d0 = start // CPP
        r0 = start % CPP
        len0 = jnp.minimum(CPP - r0, PW)
        len1 = PW - len0

        def run_segment(d, c0, steps):
            pltpu.sync_copy(tT_hbm.at[d], plane_v)
            lane = lax.iota(jnp.int32, L)
            plane_v[pl.ds(0, L)] = jnp.where(
                lane == PAD, 0.0, plane_v[pl.ds(0, L)]
            )

            def start_idx(j, b):
                c = c0 + j
                srow = c // NB
                bcol = (c % NB) * CH
                pltpu.make_async_copy(
                    xT_hbm.at[srow, pl.ds(bcol, CH)], ibufs[b], isems[b]
                ).start()

            def wait_idx(b):
                pltpu.make_async_copy(
                    xT_hbm.at[0, pl.ds(0, CH)], ibufs[b], isems[b]
                ).wait()

            def start_out(j, b):
                c = c0 + j
                srow = c // NB
                bcol = (c % NB) * CH
                pltpu.make_async_copy(
                    obufs[b], out_hbm.at[d, srow, pl.ds(bcol, CH)], osems[b]
                ).start()

            def wait_out(b):
                pltpu.make_async_copy(
                    obufs[b], out_hbm.at[d, 0, pl.ds(0, CH)], osems[b]
                ).wait()

            for b in range(DEPTH):
                start_idx(b, b)

            def body(j):
                for b in range(DEPTH):
                    jj = j + b

                    @pl.when(jj < steps)
                    def _():
                        wait_idx(b)

                        @pl.when(jj >= DEPTH)
                        def _():
                            wait_out(b)

                        def grp(g):
                            for u in range(16):
                                off = (g + u) * L
                                iv = ibufs[b][pl.ds(off, L)]
                                obufs[b][pl.ds(off, L)] = plsc.load_gather(
                                    plane_v, [iv]
                                )

                        pl.loop(0, CH // L, step=16)(grp)
                        start_out(jj, b)

                        @pl.when(jj + DEPTH < steps)
                        def _():
                            start_idx(jj + DEPTH, b)

            pl.loop(0, steps, step=DEPTH)(body)
            for b in range(DEPTH):
                wait_out(b)

        run_segment(d0, r0, len0)

        @pl.when(len1 > 0)
        def _():
            run_segment(d0 + 1, jnp.int32(0), len1)

    return lookup


def kernel(x, table):
    B_, S = x.shape
    V, D = table.shape
    out3 = _make_lookup(V, D, S, B_)(table.T, x.T)
    return jnp.transpose(out3, (2, 1, 0))
